# baseline (speedup 1.0000x reference)
import jax
import jax.numpy as jnp
import numpy as np
from jax.experimental import pallas as pl
from jax.experimental.pallas import tpu as pltpu

_DIM = 64
_HALF = _DIM // 2
_MASK_TOKEN_ID = -10.0
_LANES = 128
_POS_PER_ROW = _LANES // _DIM
_BLOCK_ROWS = 2048


def _rope_body(x_ref, f_ref, p_ref, o_ref):
    xa = x_ref[...]
    rows = xa.shape[0]
    x0 = jnp.broadcast_to(xa[:, 0:1], (rows, _LANES))
    x1 = jnp.broadcast_to(xa[:, 1:2], (rows, _LANES))
    lane = jax.lax.broadcasted_iota(jnp.int32, (rows, _LANES), 1)
    xb = jnp.where(lane < _DIM, x0, x1)
    angle = xb * f_ref[...] + p_ref[...]
    out = jnp.sin(angle)
    o_ref[...] = jnp.where(xb == _MASK_TOKEN_ID, jnp.float32(0.0), out)


def kernel(x, inv_freq):
    b, s = x.shape
    n_pos = b * s
    n_rows = n_pos // _POS_PER_ROW
    x2 = x.reshape(n_rows, _POS_PER_ROW)

    freq = jnp.tile(inv_freq, _LANES // _HALF).reshape(1, _LANES)
    lane = np.arange(_LANES)
    phase = jnp.asarray(
        np.where((lane % _DIM) >= _HALF, np.float32(np.pi / 2), np.float32(0.0)),
        dtype=jnp.float32,
    ).reshape(1, _LANES)

    grid = (n_rows // _BLOCK_ROWS,)
    out2d = pl.pallas_call(
        _rope_body,
        grid=grid,
        in_specs=[
            pl.BlockSpec((_BLOCK_ROWS, _POS_PER_ROW), lambda i: (i, 0)),
            pl.BlockSpec((1, _LANES), lambda i: (0, 0)),
            pl.BlockSpec((1, _LANES), lambda i: (0, 0)),
        ],
        out_specs=pl.BlockSpec((_BLOCK_ROWS, _LANES), lambda i: (i, 0)),
        out_shape=jax.ShapeDtypeStruct((n_rows, _LANES), jnp.float32),
        compiler_params=pltpu.CompilerParams(
            dimension_semantics=("arbitrary",),
        ),
    )(x2, freq, phase)
    return out2d.reshape(b, s, _DIM)
